# baseline (speedup 1.0000x reference)
import functools

import jax
import jax.numpy as jnp
from jax import lax
from jax.experimental import pallas as pl
from jax.experimental.pallas import tpu as pltpu
from jax.experimental.pallas import tpu_sc as plsc

N = 10000
D = 128
DEG = 32
LANES = 16

_NC = 2
_NS = 16
_NW = _NC * _NS

_NP = 10240
_PER_W = _NP // _NW
_CHUNK = 4
_NIDX = _CHUNK * DEG
_CHUNKS = _PER_W // _CHUNK


_ZROWS = 32


def _make_segsum():
    mesh = plsc.VectorSubcoreMesh(core_axis_name="c", subcore_axis_name="s")

    def body(table_hbm, idx_hbm, out_hbm, idx_v, rows_v, zero_v, dst_v,
             acc_sh, gsem):
        cid = lax.axis_index("c")
        sid = lax.axis_index("s")
        wid = sid * _NC + cid
        base = wid * _PER_W
        my0 = sid * _PER_W

        for a in range(_ZROWS):
            for s in range(D // LANES):
                zero_v[a, pl.ds(s * LANES, LANES)] = jnp.zeros(
                    (LANES,), jnp.float32)

        def zero_chunk(z, carry):
            pltpu.sync_copy(zero_v, acc_sh.at[pl.ds(my0 + z * _ZROWS, _ZROWS)])
            return carry

        lax.fori_loop(0, _PER_W // _ZROWS, zero_chunk, 0)

        def chunk(c, carry):
            atom0 = base + c * _CHUNK
            slot0 = my0 + c * _CHUNK
            for j in range(_NIDX // LANES):
                dst_v[pl.ds(j * LANES, LANES)] = jnp.full(
                    (LANES,), (j * LANES) // DEG, jnp.int32) + slot0
            pltpu.sync_copy(idx_hbm.at[pl.ds(atom0 * DEG, _NIDX)], idx_v)
            pltpu.async_copy(table_hbm.at[idx_v], rows_v, gsem).wait()
            pltpu.sync_copy(rows_v, acc_sh.at[dst_v], add=True)
            return carry

        lax.fori_loop(0, _CHUNKS, chunk, 0)
        pltpu.sync_copy(acc_sh.at[pl.ds(my0, _PER_W)],
                        out_hbm.at[pl.ds(base, _PER_W)])

    return pl.kernel(
        body,
        out_type=jax.ShapeDtypeStruct((_NP, D), jnp.float32),
        mesh=mesh,
        scratch_types=[
            pltpu.VMEM((_NIDX,), jnp.int32),
            pltpu.VMEM((_NIDX, D), jnp.float32),
            pltpu.VMEM((_ZROWS, D), jnp.float32),
            pltpu.VMEM((_NIDX,), jnp.int32),
            pltpu.VMEM_SHARED((_NS * _PER_W, D), jnp.float32),
            pltpu.SemaphoreType.DMA,
        ],
    )


_BR = 512


def _mm2_relu_body(x1, x2, w1, w2, o):
    acc = jnp.dot(x1[...], w1[...], preferred_element_type=jnp.float32)
    acc += jnp.dot(x2[...], w2[...], preferred_element_type=jnp.float32)
    o[...] = jnp.maximum(acc, 0.0)


def _mm2_relu(x1, x2, w1, w2):
    return pl.pallas_call(
        _mm2_relu_body,
        grid=(_NP // _BR,),
        in_specs=[
            pl.BlockSpec((_BR, D), lambda i: (i, 0)),
            pl.BlockSpec((_BR, D), lambda i: (i, 0)),
            pl.BlockSpec((D, D), lambda i: (0, 0)),
            pl.BlockSpec((D, D), lambda i: (0, 0)),
        ],
        out_specs=pl.BlockSpec((_BR, D), lambda i: (i, 0)),
        out_shape=jax.ShapeDtypeStruct((_NP, D), jnp.float32),
    )(x1, x2, w1, w2)


def _resid_mm_relu_body(r, x, w, o):
    acc = jnp.dot(x[...], w[...], preferred_element_type=jnp.float32)
    o[...] = jnp.maximum(r[...] + acc, 0.0)


def _resid_mm_relu(r, x, w):
    return pl.pallas_call(
        _resid_mm_relu_body,
        grid=(_NP // _BR,),
        in_specs=[
            pl.BlockSpec((_BR, D), lambda i: (i, 0)),
            pl.BlockSpec((_BR, D), lambda i: (i, 0)),
            pl.BlockSpec((D, D), lambda i: (0, 0)),
        ],
        out_specs=pl.BlockSpec((_BR, D), lambda i: (i, 0)),
        out_shape=jax.ShapeDtypeStruct((_NP, D), jnp.float32),
    )(r, x, w)


def kernel(f_atoms, f_bonds, a2b, b2a, b2revb, a_scope, b_scope, a2a,
           features_batch, W_i, W_h1, W_h2, W_o):
    del b2a, b2revb, a_scope, b_scope, features_batch
    pad_n = _NP - N
    f_atoms_p = jnp.pad(f_atoms, ((0, pad_n), (0, 0)))
    a2b_flat = jnp.pad(a2b.reshape(-1), (0, pad_n * DEG))
    a2a_flat = jnp.pad(a2a.reshape(-1), (0, pad_n * DEG))

    segsum = _make_segsum()

    bond_agg = segsum(f_bonds, a2b_flat)
    input_atom = _mm2_relu(f_atoms_p, bond_agg, W_i[:D], W_i[D:])
    agg1 = segsum(input_atom, a2a_flat)
    message = _resid_mm_relu(input_atom, agg1, W_h1)
    agg2 = segsum(message, a2a_flat)
    message = _resid_mm_relu(input_atom, agg2, W_h2)
    atom_output = _mm2_relu(f_atoms_p, message, W_o[:D], W_o[D:])
    return atom_output[:N]


# speedup vs baseline: 1.1840x; 1.1840x over previous
import functools

import jax
import jax.numpy as jnp
from jax import lax
from jax.experimental import pallas as pl
from jax.experimental.pallas import tpu as pltpu
from jax.experimental.pallas import tpu_sc as plsc

N = 10000
D = 128
DEG = 32
LANES = 16

_NC = 2
_NS = 16
_NW = _NC * _NS

_NP = 10240
_PER_W = _NP // _NW
_CHUNK = 4
_NIDX = _CHUNK * DEG
_CHUNKS = _PER_W // _CHUNK


_ZROWS = 32


def _make_segsum():
    mesh = plsc.VectorSubcoreMesh(core_axis_name="c", subcore_axis_name="s")

    def body(table_hbm, idx_hbm, out_hbm, idx_v, rows0, rows1, zero_v, dst_v,
             acc_sh, gsem0, gsem1):
        cid = lax.axis_index("c")
        sid = lax.axis_index("s")
        wid = sid * _NC + cid
        base = wid * _PER_W
        my0 = sid * _PER_W
        rows = (rows0, rows1)
        gsem = (gsem0, gsem1)

        def gather_src(c):
            return table_hbm.at[idx_v.at[pl.ds(c * _NIDX, _NIDX)]]

        pltpu.sync_copy(idx_hbm.at[pl.ds(base * DEG, _PER_W * DEG)], idx_v)

        for a in range(_ZROWS):
            for s in range(D // LANES):
                zero_v[a, pl.ds(s * LANES, LANES)] = jnp.zeros(
                    (LANES,), jnp.float32)

        def zero_chunk(z, carry):
            pltpu.sync_copy(zero_v, acc_sh.at[pl.ds(my0 + z * _ZROWS, _ZROWS)])
            return carry

        lax.fori_loop(0, _PER_W // _ZROWS, zero_chunk, 0)

        pltpu.async_copy(gather_src(0), rows[0], gsem[0])

        def pair(i, carry):
            for b in range(2):
                c = 2 * i + b

                @pl.when(c + 1 < _CHUNKS)
                def _():
                    pltpu.async_copy(gather_src(c + 1), rows[1 - b],
                                     gsem[1 - b])

                pltpu.make_async_copy(gather_src(c), rows[b], gsem[b]).wait()
                slot0 = my0 + c * _CHUNK
                for j in range(_NIDX // LANES):
                    dst_v[pl.ds(j * LANES, LANES)] = jnp.full(
                        (LANES,), (j * LANES) // DEG, jnp.int32) + slot0
                pltpu.sync_copy(rows[b], acc_sh.at[dst_v], add=True)
            return carry

        lax.fori_loop(0, _CHUNKS // 2, pair, 0)
        pltpu.sync_copy(acc_sh.at[pl.ds(my0, _PER_W)],
                        out_hbm.at[pl.ds(base, _PER_W)])

    return pl.kernel(
        body,
        out_type=jax.ShapeDtypeStruct((_NP, D), jnp.float32),
        mesh=mesh,
        scratch_types=[
            pltpu.VMEM((_PER_W * DEG,), jnp.int32),
            pltpu.VMEM((_NIDX, D), jnp.float32),
            pltpu.VMEM((_NIDX, D), jnp.float32),
            pltpu.VMEM((_ZROWS, D), jnp.float32),
            pltpu.VMEM((_NIDX,), jnp.int32),
            pltpu.VMEM_SHARED((_NS * _PER_W, D), jnp.float32),
            pltpu.SemaphoreType.DMA,
            pltpu.SemaphoreType.DMA,
        ],
    )


_BR = 512


def _mm2_relu_body(x1, x2, w1, w2, o):
    acc = jnp.dot(x1[...], w1[...], preferred_element_type=jnp.float32)
    acc += jnp.dot(x2[...], w2[...], preferred_element_type=jnp.float32)
    o[...] = jnp.maximum(acc, 0.0)


def _mm2_relu(x1, x2, w1, w2):
    return pl.pallas_call(
        _mm2_relu_body,
        grid=(_NP // _BR,),
        in_specs=[
            pl.BlockSpec((_BR, D), lambda i: (i, 0)),
            pl.BlockSpec((_BR, D), lambda i: (i, 0)),
            pl.BlockSpec((D, D), lambda i: (0, 0)),
            pl.BlockSpec((D, D), lambda i: (0, 0)),
        ],
        out_specs=pl.BlockSpec((_BR, D), lambda i: (i, 0)),
        out_shape=jax.ShapeDtypeStruct((_NP, D), jnp.float32),
    )(x1, x2, w1, w2)


def _resid_mm_relu_body(r, x, w, o):
    acc = jnp.dot(x[...], w[...], preferred_element_type=jnp.float32)
    o[...] = jnp.maximum(r[...] + acc, 0.0)


def _resid_mm_relu(r, x, w):
    return pl.pallas_call(
        _resid_mm_relu_body,
        grid=(_NP // _BR,),
        in_specs=[
            pl.BlockSpec((_BR, D), lambda i: (i, 0)),
            pl.BlockSpec((_BR, D), lambda i: (i, 0)),
            pl.BlockSpec((D, D), lambda i: (0, 0)),
        ],
        out_specs=pl.BlockSpec((_BR, D), lambda i: (i, 0)),
        out_shape=jax.ShapeDtypeStruct((_NP, D), jnp.float32),
    )(r, x, w)


def kernel(f_atoms, f_bonds, a2b, b2a, b2revb, a_scope, b_scope, a2a,
           features_batch, W_i, W_h1, W_h2, W_o):
    del b2a, b2revb, a_scope, b_scope, features_batch
    pad_n = _NP - N
    f_atoms_p = jnp.pad(f_atoms, ((0, pad_n), (0, 0)))
    a2b_flat = jnp.pad(a2b.reshape(-1), (0, pad_n * DEG))
    a2a_flat = jnp.pad(a2a.reshape(-1), (0, pad_n * DEG))

    segsum = _make_segsum()

    bond_agg = segsum(f_bonds, a2b_flat)
    input_atom = _mm2_relu(f_atoms_p, bond_agg, W_i[:D], W_i[D:])
    agg1 = segsum(input_atom, a2a_flat)
    message = _resid_mm_relu(input_atom, agg1, W_h1)
    agg2 = segsum(message, a2a_flat)
    message = _resid_mm_relu(input_atom, agg2, W_h2)
    atom_output = _mm2_relu(f_atoms_p, message, W_o[:D], W_o[D:])
    return atom_output[:N]


# speedup vs baseline: 2.2223x; 1.8769x over previous
import functools

import jax
import jax.numpy as jnp
from jax import lax
from jax.experimental import pallas as pl
from jax.experimental.pallas import tpu as pltpu
from jax.experimental.pallas import tpu_sc as plsc

N = 10000
D = 128
DEG = 32
LANES = 16

_NC = 2
_NS = 16
_NW = _NC * _NS

_NP = 10240
_PER_W = _NP // _NW
_GI = 128


def _make_segsum(stage_table: bool, chunk: int):
    mesh = plsc.VectorSubcoreMesh(core_axis_name="c", subcore_axis_name="s")
    tab_rows = 10112 if stage_table else 8
    _CHUNK = chunk
    _NIDX = _CHUNK * DEG
    _CHUNKS = _PER_W // _CHUNK
    n_g = _NIDX // _GI

    def body(table_hbm, idx_hbm, out_hbm, idx_v, rows0, rows1, zero_v,
             dst0, dst1, acc_sh, tab_sh, gsem0, gsem1, osem0, osem1):
        cid = lax.axis_index("c")
        sid = lax.axis_index("s")
        wid = sid * _NC + cid
        base = wid * _PER_W
        rows = (rows0, rows1)
        dst = (dst0, dst1)
        gsem = (gsem0, gsem1)
        osem = (osem0, osem1)

        if stage_table:
            srows = tab_rows // _NS
            pltpu.sync_copy(table_hbm.at[pl.ds(sid * srows, srows)],
                            tab_sh.at[pl.ds(sid * srows, srows)])
            table = tab_sh
        else:
            table = table_hbm

        def gather(c, b):
            for h in range(n_g):
                pltpu.async_copy(
                    table.at[idx_v.at[pl.ds(c * _NIDX + h * _GI, _GI)]],
                    rows[b].at[pl.ds(h * _GI, _GI)], gsem[b])

        def gather_wait(c, b):
            for h in range(n_g):
                pltpu.make_async_copy(
                    table.at[idx_v.at[pl.ds(c * _NIDX + h * _GI, _GI)]],
                    rows[b].at[pl.ds(h * _GI, _GI)], gsem[b]).wait()

        def slot(b):
            return (sid * 2 + b) * _CHUNK

        def out_copy(c, b):
            return pltpu.make_async_copy(
                acc_sh.at[pl.ds(slot(b), _CHUNK)],
                out_hbm.at[pl.ds(base + c * _CHUNK, _CHUNK)], osem[b])

        pltpu.sync_copy(idx_hbm.at[pl.ds(base * DEG, _PER_W * DEG)], idx_v)

        for a in range(_CHUNK):
            for s in range(D // LANES):
                zero_v[a, pl.ds(s * LANES, LANES)] = jnp.zeros(
                    (LANES,), jnp.float32)
        for b in range(2):
            for j in range(_NIDX // LANES):
                dst[b][pl.ds(j * LANES, LANES)] = jnp.full(
                    (LANES,), (j * LANES) // DEG, jnp.int32) + slot(b)

        if stage_table:
            plsc.subcore_barrier()

        gather(0, 0)

        def pair(i, carry):
            for b in range(2):
                c = 2 * i + b

                @pl.when(c + 1 < _CHUNKS)
                def _():
                    gather(c + 1, 1 - b)

                @pl.when(c >= 2)
                def _():
                    out_copy(c - 2, b).wait()

                pltpu.sync_copy(zero_v, acc_sh.at[pl.ds(slot(b), _CHUNK)])
                gather_wait(c, b)
                pltpu.sync_copy(rows[b], acc_sh.at[dst[b]], add=True)
                out_copy(c, b).start()
            return carry

        lax.fori_loop(0, _CHUNKS // 2, pair, 0)
        out_copy(_CHUNKS - 2, 0).wait()
        out_copy(_CHUNKS - 1, 1).wait()

    return pl.kernel(
        body,
        out_type=jax.ShapeDtypeStruct((_NP, D), jnp.float32),
        mesh=mesh,
        scratch_types=[
            pltpu.VMEM((_PER_W * DEG,), jnp.int32),
            pltpu.VMEM((_NIDX, D), jnp.float32),
            pltpu.VMEM((_NIDX, D), jnp.float32),
            pltpu.VMEM((_CHUNK, D), jnp.float32),
            pltpu.VMEM((_NIDX,), jnp.int32),
            pltpu.VMEM((_NIDX,), jnp.int32),
            pltpu.VMEM_SHARED((_NS * 2 * _CHUNK, D), jnp.float32),
            pltpu.VMEM_SHARED((tab_rows, D), jnp.float32),
            pltpu.SemaphoreType.DMA,
            pltpu.SemaphoreType.DMA,
            pltpu.SemaphoreType.DMA,
            pltpu.SemaphoreType.DMA,
        ],
    )


_BR = 512


def _mm2_relu_body(x1, x2, w1, w2, o):
    acc = jnp.dot(x1[...], w1[...], preferred_element_type=jnp.float32)
    acc += jnp.dot(x2[...], w2[...], preferred_element_type=jnp.float32)
    o[...] = jnp.maximum(acc, 0.0)


def _mm2_relu(x1, x2, w1, w2):
    return pl.pallas_call(
        _mm2_relu_body,
        grid=(_NP // _BR,),
        in_specs=[
            pl.BlockSpec((_BR, D), lambda i: (i, 0)),
            pl.BlockSpec((_BR, D), lambda i: (i, 0)),
            pl.BlockSpec((D, D), lambda i: (0, 0)),
            pl.BlockSpec((D, D), lambda i: (0, 0)),
        ],
        out_specs=pl.BlockSpec((_BR, D), lambda i: (i, 0)),
        out_shape=jax.ShapeDtypeStruct((_NP, D), jnp.float32),
    )(x1, x2, w1, w2)


def _resid_mm_relu_body(r, x, w, o):
    acc = jnp.dot(x[...], w[...], preferred_element_type=jnp.float32)
    o[...] = jnp.maximum(r[...] + acc, 0.0)


def _resid_mm_relu(r, x, w):
    return pl.pallas_call(
        _resid_mm_relu_body,
        grid=(_NP // _BR,),
        in_specs=[
            pl.BlockSpec((_BR, D), lambda i: (i, 0)),
            pl.BlockSpec((_BR, D), lambda i: (i, 0)),
            pl.BlockSpec((D, D), lambda i: (0, 0)),
        ],
        out_specs=pl.BlockSpec((_BR, D), lambda i: (i, 0)),
        out_shape=jax.ShapeDtypeStruct((_NP, D), jnp.float32),
    )(r, x, w)


def kernel(f_atoms, f_bonds, a2b, b2a, b2revb, a_scope, b_scope, a2a,
           features_batch, W_i, W_h1, W_h2, W_o):
    del b2a, b2revb, a_scope, b_scope, features_batch
    pad_n = _NP - N
    f_atoms_p = jnp.pad(f_atoms, ((0, pad_n), (0, 0)))
    a2b_flat = jnp.pad(a2b.reshape(-1), (0, pad_n * DEG))
    a2a_flat = jnp.pad(a2a.reshape(-1), (0, pad_n * DEG))

    segsum_hbm = _make_segsum(stage_table=False, chunk=8)
    segsum_sp = _make_segsum(stage_table=True, chunk=4)

    bond_agg = segsum_hbm(f_bonds, a2b_flat)
    input_atom = _mm2_relu(f_atoms_p, bond_agg, W_i[:D], W_i[D:])
    agg1 = segsum_sp(input_atom, a2a_flat)
    message = _resid_mm_relu(input_atom, agg1, W_h1)
    agg2 = segsum_sp(message, a2a_flat)
    message = _resid_mm_relu(input_atom, agg2, W_h2)
    atom_output = _mm2_relu(f_atoms_p, message, W_o[:D], W_o[D:])
    return atom_output[:N]
